# baseline (speedup 1.0000x reference)
import functools

import jax
import jax.numpy as jnp
from jax import lax
from jax.experimental import pallas as pl
from jax.experimental.pallas import tpu as pltpu
from jax.experimental.pallas import tpu_sc as plsc

NUM_NODES_IN = 10000
NUM_EDGES_IN = 320000
EDGE_DIM = 16
HIDDEN_DIM = 128

NC = 2
NS = 16
NW = NC * NS

N_PAD = 10240
ROWS_PER_TILE = N_PAD // NS
SUB = 128
N_BLK = NUM_EDGES_IN // SUB
BLK_PER_W = N_BLK // NW
EXTRA_W = N_BLK % NW
CHUNK_BLKS = 13
N_CHUNKS = BLK_PER_W // CHUNK_BLKS

PACK_CB = 3200


def _tc_pack_body(x_ref, o_ref):
    x = x_ref[...]
    o_ref[...] = (x.reshape(16, PACK_CB // 8, 8)
                  .transpose(1, 2, 0).reshape(PACK_CB // 8, 128))


def _tc_pack(eT):
    return pl.pallas_call(
        _tc_pack_body,
        grid=(NUM_EDGES_IN // PACK_CB,),
        in_specs=[pl.BlockSpec((16, PACK_CB), lambda i: (0, i))],
        out_specs=pl.BlockSpec((PACK_CB // 8, 128), lambda i: (i, 0)),
        out_shape=jax.ShapeDtypeStruct((NUM_EDGES_IN // 8, 128), jnp.float32),
    )(eT)


def _sc_segment_sum(edge_rm, ei3):
    mesh = plsc.VectorSubcoreMesh(core_axis_name="c", subcore_axis_name="s")

    @functools.partial(
        pl.kernel,
        mesh=mesh,
        compiler_params=pltpu.CompilerParams(use_tc_tiling_on_sc=False),
        out_type=(
            jax.ShapeDtypeStruct((NC, N_PAD, EDGE_DIM), jnp.float32),
            jax.ShapeDtypeStruct((NC * N_PAD,), jnp.float32),
        ),
        scratch_types=[
            pltpu.VMEM((CHUNK_BLKS * SUB, EDGE_DIM), jnp.float32),
            pltpu.VMEM((BLK_PER_W + 1, 2, SUB), jnp.int32),
            pltpu.VMEM((SUB,), jnp.float32),
            pltpu.VMEM((ROWS_PER_TILE, EDGE_DIM), jnp.float32),
            pltpu.VMEM((ROWS_PER_TILE,), jnp.float32),
            pltpu.VMEM_SHARED((N_PAD, EDGE_DIM), jnp.float32),
            pltpu.VMEM_SHARED((N_PAD,), jnp.float32),
        ],
    )
    def seg(edge_hbm, dst_hbm, part_out, cnt_out,
            buf, idxv, onesv, zbuf, zcnt, acc_sh, cnt_sh):
        cid = lax.axis_index("c")
        sid = lax.axis_index("s")
        wid = cid * NS + sid
        base_blk = wid * BLK_PER_W + jnp.minimum(wid, EXTRA_W)

        ones16 = jnp.full((16,), 1.0, dtype=jnp.float32)
        zeros16 = jnp.zeros((16,), dtype=jnp.float32)

        for i in range(SUB // 16):
            onesv[pl.ds(i * 16, 16)] = ones16

        def zrow(i, _):
            zbuf[i, :] = zeros16
            return 0
        lax.fori_loop(0, ROWS_PER_TILE, zrow, 0)

        def zrow1(i, _):
            zcnt[pl.ds(i * 16, 16)] = zeros16
            return 0
        lax.fori_loop(0, ROWS_PER_TILE // 16, zrow1, 0)

        row0 = sid * ROWS_PER_TILE
        pltpu.sync_copy(zbuf, acc_sh.at[pl.ds(row0, ROWS_PER_TILE)])
        pltpu.sync_copy(zcnt, cnt_sh.at[pl.ds(row0, ROWS_PER_TILE)])
        plsc.subcore_barrier()

        pltpu.sync_copy(dst_hbm.at[pl.ds(base_blk, BLK_PER_W)],
                        idxv.at[pl.ds(0, BLK_PER_W)])

        def process(nblk, blk0):
            pltpu.sync_copy(edge_hbm.at[pl.ds(blk0 * SUB, nblk * SUB)],
                            buf.at[pl.ds(0, nblk * SUB)])

            def sub_body(t, _):
                idx = idxv.at[blk0 - base_blk + t, 1]
                pltpu.sync_copy(buf.at[pl.ds(t * SUB, SUB)],
                                acc_sh.at[idx], add=True)
                pltpu.sync_copy(onesv, cnt_sh.at[idx], add=True)
                return 0
            lax.fori_loop(0, nblk, sub_body, 0)

        def chunk_body(g, _):
            process(CHUNK_BLKS, base_blk + g * CHUNK_BLKS)
            return 0
        lax.fori_loop(0, N_CHUNKS, chunk_body, 0)

        @pl.when(wid < EXTRA_W)
        def _():
            xblk = base_blk + BLK_PER_W
            pltpu.sync_copy(dst_hbm.at[pl.ds(xblk, 1)],
                            idxv.at[pl.ds(BLK_PER_W, 1)])
            process(1, xblk)

        plsc.subcore_barrier()

        pltpu.sync_copy(acc_sh.at[pl.ds(row0, ROWS_PER_TILE)], zbuf)
        pltpu.sync_copy(zbuf, part_out.at[cid, pl.ds(row0, ROWS_PER_TILE)])
        pltpu.sync_copy(cnt_sh.at[pl.ds(row0, ROWS_PER_TILE)], zcnt)
        pltpu.sync_copy(
            zcnt, cnt_out.at[pl.ds(cid * N_PAD + row0, ROWS_PER_TILE)])

    return seg(edge_rm, ei3)


def _tc_dense_body(p_ref, c_ref, w1_ref, b1_ref, w2_ref, b2_ref, o_ref):
    s = p_ref[0] + p_ref[1]
    n = c_ref[:, 0:1] + c_ref[:, 1:2]
    c = jnp.maximum(n, 1.0)
    m = jnp.dot(s, w1_ref[...], preferred_element_type=jnp.float32)
    agg = (m + n * b1_ref[...]) / c
    o_ref[...] = (
        jnp.dot(agg, w2_ref[...], preferred_element_type=jnp.float32)
        + b2_ref[...]
    )


def _tc_dense(part, cnt_t, proj_W, proj_b2, upd_W, upd_b2):
    R = 1000
    grid = (NUM_NODES_IN // R,)
    return pl.pallas_call(
        _tc_dense_body,
        grid=grid,
        in_specs=[
            pl.BlockSpec((NC, R, EDGE_DIM), lambda i: (0, i, 0)),
            pl.BlockSpec((R, NC), lambda i: (i, 0)),
            pl.BlockSpec((EDGE_DIM, HIDDEN_DIM), lambda i: (0, 0)),
            pl.BlockSpec((1, HIDDEN_DIM), lambda i: (0, 0)),
            pl.BlockSpec((HIDDEN_DIM, HIDDEN_DIM), lambda i: (0, 0)),
            pl.BlockSpec((1, HIDDEN_DIM), lambda i: (0, 0)),
        ],
        out_specs=pl.BlockSpec((R, HIDDEN_DIM), lambda i: (i, 0)),
        out_shape=jax.ShapeDtypeStruct((NUM_NODES_IN, HIDDEN_DIM),
                                       jnp.float32),
    )(part, cnt_t, proj_W, proj_b2, upd_W, upd_b2)


@jax.jit
def kernel(edge_attr, edge_index, proj_W, proj_b, upd_W, upd_b):
    edge_rm = _tc_pack(edge_attr.T).reshape(NUM_EDGES_IN, EDGE_DIM)
    ei3 = (edge_index.astype(jnp.int32)
           .reshape(2, N_BLK, SUB).transpose(1, 0, 2))
    part, cnt = _sc_segment_sum(edge_rm, ei3)
    return _tc_dense(
        part,
        cnt.reshape(NC, N_PAD).T,
        proj_W,
        proj_b.reshape(1, HIDDEN_DIM),
        upd_W,
        upd_b.reshape(1, HIDDEN_DIM),
    )


# speedup vs baseline: 1.8818x; 1.8818x over previous
import functools

import jax
import jax.numpy as jnp
from jax import lax
from jax.experimental import pallas as pl
from jax.experimental.pallas import tpu as pltpu
from jax.experimental.pallas import tpu_sc as plsc

NUM_NODES_IN = 10000
NUM_EDGES_IN = 320000
EDGE_DIM = 16
HIDDEN_DIM = 128

NC = 2
NS = 16
NW = NC * NS

N_PAD = 10240
ROWS_PER_TILE = N_PAD // NS
SUB = 128
N_BLK = NUM_EDGES_IN // SUB
BLK_PER_W = N_BLK // NW
EXTRA_W = N_BLK % NW
CHUNK_BLKS = 13
N_CHUNKS = BLK_PER_W // CHUNK_BLKS


def _sc_segment_sum(edge_attr, ei3):
    mesh = plsc.VectorSubcoreMesh(core_axis_name="c", subcore_axis_name="s")

    @functools.partial(
        pl.kernel,
        mesh=mesh,
        compiler_params=pltpu.CompilerParams(use_tc_tiling_on_sc=False),
        out_type=(
            jax.ShapeDtypeStruct((NC, N_PAD, EDGE_DIM), jnp.float32),
            jax.ShapeDtypeStruct((NC * N_PAD,), jnp.float32),
        ),
        scratch_types=[
            pltpu.VMEM((2, CHUNK_BLKS * SUB, EDGE_DIM), jnp.float32),
            pltpu.VMEM((BLK_PER_W + 1, 2, SUB), jnp.int32),
            pltpu.VMEM((SUB,), jnp.float32),
            pltpu.VMEM((ROWS_PER_TILE, EDGE_DIM), jnp.float32),
            pltpu.VMEM((ROWS_PER_TILE,), jnp.float32),
            pltpu.VMEM_SHARED((N_PAD, EDGE_DIM), jnp.float32),
            pltpu.VMEM_SHARED((N_PAD,), jnp.float32),
            pltpu.SemaphoreType.DMA,
            pltpu.SemaphoreType.DMA,
            pltpu.SemaphoreType.DMA,
        ],
    )
    def seg(edge_hbm, dst_hbm, part_out, cnt_out,
            buf2, idxv, onesv, zbuf, zcnt, acc_sh, cnt_sh,
            dsem, csem, ssem):
        cid = lax.axis_index("c")
        sid = lax.axis_index("s")
        wid = cid * NS + sid
        base_blk = wid * BLK_PER_W + jnp.minimum(wid, EXTRA_W)

        ones16 = jnp.full((16,), 1.0, dtype=jnp.float32)
        zeros16 = jnp.zeros((16,), dtype=jnp.float32)

        for i in range(SUB // 16):
            onesv[pl.ds(i * 16, 16)] = ones16

        def zrow(i, _):
            zbuf[i, :] = zeros16
            return 0
        lax.fori_loop(0, ROWS_PER_TILE, zrow, 0)

        def zrow1(i, _):
            zcnt[pl.ds(i * 16, 16)] = zeros16
            return 0
        lax.fori_loop(0, ROWS_PER_TILE // 16, zrow1, 0)

        row0 = sid * ROWS_PER_TILE
        pltpu.sync_copy(zbuf, acc_sh.at[pl.ds(row0, ROWS_PER_TILE)])
        pltpu.sync_copy(zcnt, cnt_sh.at[pl.ds(row0, ROWS_PER_TILE)])
        plsc.subcore_barrier()

        pltpu.sync_copy(dst_hbm.at[pl.ds(base_blk, BLK_PER_W)],
                        idxv.at[pl.ds(0, BLK_PER_W)])

        def stage(g, sl):
            blk0 = base_blk + g * CHUNK_BLKS
            return pltpu.async_copy(
                edge_hbm.at[pl.ds(blk0 * SUB, CHUNK_BLKS * SUB)],
                buf2.at[sl], ssem)

        def fire(g, sl):
            buf = buf2.at[sl]

            def issue(t, _):
                idx = idxv.at[g * CHUNK_BLKS + t, 1]
                pltpu.async_copy(buf.at[pl.ds(t * SUB, SUB)],
                                 acc_sh.at[idx], dsem, add=True)
                pltpu.async_copy(onesv, cnt_sh.at[idx], csem, add=True)
                return 0
            lax.fori_loop(0, CHUNK_BLKS, issue, 0)

        def drain(g, sl):
            buf = buf2.at[sl]

            def dwait(t, _):
                idx = idxv.at[g * CHUNK_BLKS + t, 1]
                pltpu.make_async_copy(buf.at[pl.ds(t * SUB, SUB)],
                                      acc_sh.at[idx], dsem).wait()
                pltpu.make_async_copy(onesv, cnt_sh.at[idx], csem).wait()
                return 0
            lax.fori_loop(0, CHUNK_BLKS, dwait, 0)

        stage(0, 0).wait()

        def chunk_body(g, _):
            sl = lax.rem(g, 2)
            fire(g, sl)

            @pl.when(g + 1 < N_CHUNKS)
            def _():
                stage(g + 1, 1 - sl)

            drain(g, sl)

            @pl.when(g + 1 < N_CHUNKS)
            def _():
                pltpu.make_async_copy(
                    edge_hbm.at[pl.ds(base_blk * SUB, CHUNK_BLKS * SUB)],
                    buf2.at[1 - sl], ssem).wait()
            return 0
        lax.fori_loop(0, N_CHUNKS, chunk_body, 0)

        @pl.when(wid < EXTRA_W)
        def _():
            xblk = base_blk + BLK_PER_W
            pltpu.sync_copy(dst_hbm.at[pl.ds(xblk, 1)],
                            idxv.at[pl.ds(BLK_PER_W, 1)])
            pltpu.sync_copy(edge_hbm.at[pl.ds(xblk * SUB, SUB)],
                            buf2.at[0, pl.ds(0, SUB)])
            idx = idxv.at[BLK_PER_W, 1]
            pltpu.sync_copy(buf2.at[0, pl.ds(0, SUB)], acc_sh.at[idx],
                            add=True)
            pltpu.sync_copy(onesv, cnt_sh.at[idx], add=True)

        plsc.subcore_barrier()

        pltpu.sync_copy(acc_sh.at[pl.ds(row0, ROWS_PER_TILE)], zbuf)
        pltpu.sync_copy(zbuf, part_out.at[cid, pl.ds(row0, ROWS_PER_TILE)])
        pltpu.sync_copy(cnt_sh.at[pl.ds(row0, ROWS_PER_TILE)], zcnt)
        pltpu.sync_copy(
            zcnt, cnt_out.at[pl.ds(cid * N_PAD + row0, ROWS_PER_TILE)])

    return seg(edge_attr, ei3)


def _tc_dense_body(p_ref, c_ref, w1_ref, b1_ref, w2_ref, b2_ref, o_ref):
    s = p_ref[0] + p_ref[1]
    n = c_ref[:, 0:1] + c_ref[:, 1:2]
    c = jnp.maximum(n, 1.0)
    m = jnp.dot(s, w1_ref[...], preferred_element_type=jnp.float32)
    agg = (m + n * b1_ref[...]) / c
    o_ref[...] = (
        jnp.dot(agg, w2_ref[...], preferred_element_type=jnp.float32)
        + b2_ref[...]
    )


def _tc_dense(part, cnt_t, proj_W, proj_b2, upd_W, upd_b2):
    R = 1000
    grid = (NUM_NODES_IN // R,)
    return pl.pallas_call(
        _tc_dense_body,
        grid=grid,
        in_specs=[
            pl.BlockSpec((NC, R, EDGE_DIM), lambda i: (0, i, 0)),
            pl.BlockSpec((R, NC), lambda i: (i, 0)),
            pl.BlockSpec((EDGE_DIM, HIDDEN_DIM), lambda i: (0, 0)),
            pl.BlockSpec((1, HIDDEN_DIM), lambda i: (0, 0)),
            pl.BlockSpec((HIDDEN_DIM, HIDDEN_DIM), lambda i: (0, 0)),
            pl.BlockSpec((1, HIDDEN_DIM), lambda i: (0, 0)),
        ],
        out_specs=pl.BlockSpec((R, HIDDEN_DIM), lambda i: (i, 0)),
        out_shape=jax.ShapeDtypeStruct((NUM_NODES_IN, HIDDEN_DIM),
                                       jnp.float32),
    )(part, cnt_t, proj_W, proj_b2, upd_W, upd_b2)


@jax.jit
def kernel(edge_attr, edge_index, proj_W, proj_b, upd_W, upd_b):
    ei3 = (edge_index.astype(jnp.int32)
           .reshape(2, N_BLK, SUB).transpose(1, 0, 2))
    part, cnt = _sc_segment_sum(edge_attr, ei3)
    return _tc_dense(
        part,
        cnt.reshape(NC, N_PAD).T,
        proj_W,
        proj_b.reshape(1, HIDDEN_DIM),
        upd_W,
        upd_b.reshape(1, HIDDEN_DIM),
    )
